# baseline (speedup 1.0000x reference)
import functools

import jax
import jax.numpy as jnp
from jax import lax
from jax.experimental import pallas as pl
from jax.experimental.pallas import tpu as pltpu
from jax.experimental.pallas import tpu_sc as plsc

HID = 128
N_E = 320000
N_C5 = 25000
N_C6 = 30000

CH_E = 256
NCH_E = 216
GB_E = 128
CH_V = 320
NCH_V = 1000
GB_V = 64
NW = 32


def _rup(x, m):
    return (x + m - 1) // m * m


def _sread(po_ref, i):
    return po_ref[pl.ds(i, 16)][0]


def _make_sc_scatter(nt, c, ch_rows, nch, gb, m_pad, npo):
    chp = ch_rows + 16
    trips = (nch + NW - 1) // NW
    mesh = plsc.VectorSubcoreMesh(core_axis_name="c", subcore_axis_name="s")

    @functools.partial(
        pl.kernel,
        out_type=jax.ShapeDtypeStruct((nch * ch_rows, c), jnp.float32),
        mesh=mesh,
        scratch_types=[
            pltpu.VMEM((npo,), jnp.int32),
            pltpu.VMEM((gb,), jnp.int32),
            pltpu.VMEM((gb,), jnp.int32),
            pltpu.VMEM((gb + 16,), jnp.int32),
            pltpu.VMEM((gb + 16,), jnp.int32),
            pltpu.VMEM((gb, c), jnp.float32),
            pltpu.VMEM((gb, c), jnp.float32),
            pltpu.VMEM((chp, c), jnp.float32),
            pltpu.SemaphoreType.DMA,
            pltpu.SemaphoreType.DMA,
            pltpu.SemaphoreType.DMA,
            pltpu.SemaphoreType.DMA,
            pltpu.SemaphoreType.DMA,
            pltpu.SemaphoreType.DMA,
        ],
    )
    def scatter_kernel(table, src_hbm, l2_hbm, po_hbm, out,
                       po_v, src0, src1, lb0, lb1, rows0, rows1, acc,
                       ss0, ss1, sl0, sl1, sr0, sr1):
        cid = lax.axis_index("c")
        sid = lax.axis_index("s")
        wid = sid * 2 + cid
        srcb = (src0, src1)
        lbb = (lb0, lb1)
        rowsb = (rows0, rows1)
        ssem = (ss0, ss1)
        lsem = (sl0, sl1)
        rsem = (sr0, sr1)
        zvec = jnp.zeros((16,), jnp.float32)

        pltpu.sync_copy(po_hbm, po_v)

        def process(chi):
            p0 = _sread(po_v, chi)
            p1 = _sread(po_v, chi + 1)
            ncb = (p1 - p0) // gb

            def zrow(r, carry):
                for cc in range(c // 16):
                    acc[r, pl.ds(cc * 16, 16)] = zvec
                return carry

            lax.fori_loop(0, chp, zrow, 0)

            def fetch(n, q):
                ofs = pl.multiple_of(p0 + n * gb, gb)
                pltpu.async_copy(src_hbm.at[pl.ds(ofs, gb)], srcb[q],
                                 ssem[q])
                pltpu.async_copy(l2_hbm.at[pl.ds(ofs, gb)],
                                 lbb[q].at[pl.ds(0, gb)], lsem[q])

            def fetch_wait(q):
                pltpu.make_async_copy(src_hbm.at[pl.ds(0, gb)], srcb[q],
                                      ssem[q]).wait()
                pltpu.make_async_copy(l2_hbm.at[pl.ds(0, gb)],
                                      lbb[q].at[pl.ds(0, gb)],
                                      lsem[q]).wait()

            @pl.when(ncb > 0)
            def _():
                fetch(0, 0)
                fetch_wait(0)
                pltpu.async_copy(table.at[srcb[0]], rowsb[0], rsem[0])

                def batch_for(g, p):
                    q = 1 - p
                    nxt = g + 1

                    @pl.when(nxt < ncb)
                    def _():
                        fetch(nxt, q)
                    pltpu.make_async_copy(table.at[srcb[p]], rowsb[p],
                                          rsem[p]).wait()

                    @pl.when(nxt < ncb)
                    def _():
                        fetch_wait(q)
                        pltpu.async_copy(table.at[srcb[q]], rowsb[q],
                                         rsem[q])
                    def add_row(r, carry3):
                        row = _sread(lbb[p], r)
                        for cc in range(c // 16):
                            sl = pl.ds(cc * 16, 16)
                            acc[row, sl] += rowsb[p][r, sl]
                        return carry3

                    lax.fori_loop(0, gb, add_row, 0)

                def batch_body(g, carry2):
                    pr = lax.rem(g, 2)

                    @pl.when(pr == 0)
                    def _():
                        batch_for(g, 0)

                    @pl.when(pr == 1)
                    def _():
                        batch_for(g, 1)
                    return carry2

                lax.fori_loop(0, ncb, batch_body, 0)

            pltpu.sync_copy(
                acc.at[pl.ds(0, ch_rows)],
                out.at[pl.ds(pl.multiple_of(chi * ch_rows, 8), ch_rows)])

        def chunk_body(i, carry):
            chi = wid + i * NW

            @pl.when(chi < nch)
            def _():
                process(chi)
            return carry

        lax.fori_loop(0, trips, chunk_body, 0)

    return scatter_kernel


def _build_entries(srcs, dsts, ch_rows, nch, padb, trash_mod):
    chp = ch_rows + 16
    m = srcs.shape[0]
    order = jnp.argsort(dsts)
    srcs = srcs[order].astype(jnp.int32)
    dsts = dsts[order].astype(jnp.int32)
    chunk = dsts // ch_rows
    cnt = jnp.bincount(chunk, length=nch)
    start = jnp.concatenate([jnp.zeros((1,), jnp.int32),
                             jnp.cumsum(cnt)[:-1].astype(jnp.int32)])
    cnt_pad = _rup(cnt, padb)
    po_body = jnp.concatenate([jnp.zeros((1,), jnp.int32),
                               jnp.cumsum(cnt_pad).astype(jnp.int32)])
    m_pad = m + nch * padb
    npo = _rup(nch + 1, 16) + 16
    po = jnp.concatenate(
        [po_body, jnp.full((npo - nch - 1,), po_body[-1], jnp.int32)])
    pos = po_body[chunk] + (jnp.arange(m, dtype=jnp.int32) - start[chunk])
    j = jnp.arange(m_pad, dtype=jnp.int32)
    src_pad = jnp.zeros((m_pad,), jnp.int32).at[pos].set(srcs)
    ldst_pad = jnp.zeros((m_pad,), jnp.int32).at[pos].set(dsts % ch_rows)
    valid = jnp.zeros((m_pad,), jnp.bool_).at[pos].set(True)
    src_pad = jnp.where(valid, src_pad, j % 1024)
    ldst_pad = jnp.where(valid, ldst_pad, ch_rows + j % trash_mod)
    return src_pad, ldst_pad, po, m_pad, npo


def _proj_body(g, db, dir_ref, cyc_ref, w0_ref, wd_ref, wc_ref,
               out_ref, stat_ref):
    r = dir_ref.shape[0]
    x = dir_ref[...]
    rows = lax.broadcasted_iota(jnp.int32, (db, r), 1)
    cols = lax.broadcasted_iota(jnp.int32, (db, r), 0)
    msum = (rows // g == cols).astype(jnp.float32)
    d = jnp.dot(msum, x, preferred_element_type=jnp.float32)
    dsb = jnp.dot(msum.T, d, preferred_element_type=jnp.float32)
    out = (jnp.dot(x, w0_ref[...], preferred_element_type=jnp.float32)
           + jnp.dot(dsb, wd_ref[...], preferred_element_type=jnp.float32)
           + jnp.dot(cyc_ref[...], wc_ref[...],
                     preferred_element_type=jnp.float32))
    out_ref[...] = out
    h = out[:, 512:768]

    @pl.when(pl.program_id(0) == 0)
    def _():
        stat_ref[...] = jnp.zeros_like(stat_ref)
    stat_ref[0:1, :] += jnp.sum(h, axis=0, keepdims=True)
    stat_ref[1:2, :] += jnp.sum(h * h, axis=0, keepdims=True)


def _cycle_proj(dirx, cyc, w0, wd, wc, g, r, db):
    n = dirx.shape[0]
    grid = n // r
    body = functools.partial(_proj_body, g, db)
    return pl.pallas_call(
        body,
        grid=(grid,),
        in_specs=[
            pl.BlockSpec((r, HID), lambda i: (i, 0)),
            pl.BlockSpec((r, HID), lambda i: (i, 0)),
            pl.BlockSpec((HID, 768), lambda i: (0, 0)),
            pl.BlockSpec((HID, 768), lambda i: (0, 0)),
            pl.BlockSpec((HID, 768), lambda i: (0, 0)),
        ],
        out_specs=[
            pl.BlockSpec((r, 768), lambda i: (i, 0)),
            pl.BlockSpec((8, 256), lambda i: (0, 0)),
        ],
        out_shape=[
            jax.ShapeDtypeStruct((n, 768), jnp.float32),
            jax.ShapeDtypeStruct((8, 256), jnp.float32),
        ],
    )(dirx, cyc, w0, wd, wc)


def _h1_body(e_ref, v_ref, wa_ref, out_ref, stat_ref):
    h = jnp.dot(e_ref[...], wa_ref[...],
                preferred_element_type=jnp.float32) + v_ref[...]
    out_ref[...] = h

    @pl.when(pl.program_id(0) == 0)
    def _():
        stat_ref[...] = jnp.zeros_like(stat_ref)
    stat_ref[0:1, :] += jnp.sum(h, axis=0, keepdims=True)
    stat_ref[1:2, :] += jnp.sum(h * h, axis=0, keepdims=True)


def _edge_h1(edge, veff, wa, r):
    n = edge.shape[0]
    return pl.pallas_call(
        _h1_body,
        grid=(n // r,),
        in_specs=[
            pl.BlockSpec((r, HID), lambda i: (i, 0)),
            pl.BlockSpec((r, 256), lambda i: (i, 0)),
            pl.BlockSpec((HID, 256), lambda i: (0, 0)),
        ],
        out_specs=[
            pl.BlockSpec((r, 256), lambda i: (i, 0)),
            pl.BlockSpec((8, 256), lambda i: (0, 0)),
        ],
        out_shape=[
            jax.ShapeDtypeStruct((n, 256), jnp.float32),
            jax.ShapeDtypeStruct((8, 256), jnp.float32),
        ],
    )(edge, veff, wa)


def _nm_body(x_ref, sc_ref, sh_ref, w_ref, out_ref, stat_ref):
    y = jnp.maximum(x_ref[...] * sc_ref[0:1, :] + sh_ref[0:1, :], 0.0)
    h = jnp.dot(y, w_ref[...], preferred_element_type=jnp.float32)
    out_ref[...] = h

    @pl.when(pl.program_id(0) == 0)
    def _():
        stat_ref[...] = jnp.zeros_like(stat_ref)
    stat_ref[0:1, :] += jnp.sum(h, axis=0, keepdims=True)
    stat_ref[1:2, :] += jnp.sum(h * h, axis=0, keepdims=True)


def _norm_matmul(x, scale, shift, w, r):
    n, cin = x.shape
    cout = w.shape[1]
    return pl.pallas_call(
        _nm_body,
        grid=(n // r,),
        in_specs=[
            pl.BlockSpec((r, cin), lambda i: (i, 0)),
            pl.BlockSpec((8, cin), lambda i: (0, 0)),
            pl.BlockSpec((8, cin), lambda i: (0, 0)),
            pl.BlockSpec((cin, cout), lambda i: (0, 0)),
        ],
        out_specs=[
            pl.BlockSpec((r, cout), lambda i: (i, 0)),
            pl.BlockSpec((8, cout), lambda i: (0, 0)),
        ],
        out_shape=[
            jax.ShapeDtypeStruct((n, cout), jnp.float32),
            jax.ShapeDtypeStruct((8, cout), jnp.float32),
        ],
    )(x, scale, shift, w)


def _relu_body(x_ref, sc_ref, sh_ref, out_ref):
    out_ref[...] = jnp.maximum(
        x_ref[...] * sc_ref[0:1, :] + sh_ref[0:1, :], 0.0)


def _norm_relu(x, scale, shift, r):
    n, c = x.shape
    return pl.pallas_call(
        _relu_body,
        grid=(n // r,),
        in_specs=[
            pl.BlockSpec((r, c), lambda i: (i, 0)),
            pl.BlockSpec((8, c), lambda i: (0, 0)),
            pl.BlockSpec((8, c), lambda i: (0, 0)),
        ],
        out_specs=pl.BlockSpec((r, c), lambda i: (i, 0)),
        out_shape=jax.ShapeDtypeStruct((n, c), jnp.float32),
    )(x, scale, shift)


def _bn_affine(stat, n, gamma, beta):
    mu = stat[0, :] / n
    var = stat[1, :] / n - mu * mu
    scale = gamma / jnp.sqrt(var + 1e-5)
    shift = beta - mu * scale
    pad = jnp.zeros((8, scale.shape[0]), jnp.float32)
    return pad.at[0, :].set(scale), pad.at[0, :].set(shift)


def kernel(edge_rep, cycle5_rep, cycle6_rep, eW1, eg1, eb1, eW2, eg2, eb2,
           cW1, cg1, cb1, cW2, cg2, cb2, e2c5_src, e2c5_dst, e2c6_src,
           e2c6_dst, c2e5_src, c2e5_dst, c2e6_src, c2e6_dst):
    f32 = jnp.float32
    edge_rep = edge_rep.astype(f32)

    def fold(g):
        w0 = jnp.concatenate(
            [eW1[128:256], eW1[768:896], cW1[0:128]], axis=1)
        wd = jnp.concatenate(
            [eW1[256:384] + eW1[384:512] + g * eW1[512:640],
             eW1[896:1024] + eW1[1024:1152] + g * eW1[1152:1280],
             cW1[128:256] + cW1[256:384] + g * cW1[384:512]], axis=1)
        wc = jnp.concatenate(
            [eW1[640:768], eW1[1280:1408], cW1[512:640]], axis=1)
        return w0, wd, wc

    w0_5, wd_5, wc_5 = fold(5.0)
    w0_6, wd_6, wc_6 = fold(6.0)
    wa = eW1[0:128]

    src_e = jnp.concatenate([e2c5_src, e2c6_src]).astype(jnp.int32)
    dst_e = jnp.concatenate(
        [e2c5_dst, e2c6_dst + N_C5]).astype(jnp.int32)
    sp_e, l2_e, po_e, mpad_e, npo_e = _build_entries(
        src_e, dst_e, CH_E, NCH_E, GB_E, 16)
    sc1 = _make_sc_scatter(N_E, HID, CH_E, NCH_E, GB_E, mpad_e, npo_e)
    dir_all = sc1(edge_rep, sp_e, l2_e, po_e)
    dir5 = dir_all[:N_C5]
    dir6 = dir_all[N_C5:N_C5 + N_C6]

    p5, stat5 = _cycle_proj(dir5, cycle5_rep.astype(f32),
                            w0_5, wd_5, wc_5, 5, 1000, 200)
    p6, stat6 = _cycle_proj(dir6, cycle6_rep.astype(f32),
                            w0_6, wd_6, wc_6, 6, 1200, 200)
    pstack = jnp.concatenate(
        [p5[:, 0:256], p5[:, 256:512], p6[:, 0:256], p6[:, 256:512]], axis=0)

    b5c = N_C5
    b6b = 2 * N_C5
    b6c = 2 * N_C5 + N_C6
    s5 = c2e5_src.astype(jnp.int32)
    d5 = c2e5_dst.astype(jnp.int32)
    s6 = c2e6_src.astype(jnp.int32)
    d6 = c2e6_dst.astype(jnp.int32)
    src_v = jnp.concatenate(
        [s5, s5 + b5c, s5 + b5c, s6 + b6b, s6 + b6c, s6 + b6c])
    dst_v = jnp.concatenate(
        [d5, d5 & ~1, d5 | 1, d6, d6 & ~1, d6 | 1])
    sp_v, l2_v, po_v, mpad_v, npo_v = _build_entries(
        src_v, dst_v, CH_V, NCH_V, GB_V, 16)
    sc2 = _make_sc_scatter(2 * (N_C5 + N_C6), 256, CH_V, NCH_V, GB_V,
                           mpad_v, npo_v)
    veff = sc2(pstack, sp_v, l2_v, po_v)

    h1, stat_h1 = _edge_h1(edge_rep, veff, wa, 1000)
    sc_1, sh_1 = _bn_affine(stat_h1, float(N_E), eg1, eb1)
    h2, stat_h2 = _norm_matmul(h1, sc_1, sh_1, eW2, 1000)
    sc_2, sh_2 = _bn_affine(stat_h2, float(N_E), eg2, eb2)
    edge_out = _norm_relu(h2, sc_2, sh_2, 1000)

    h5 = p5[:, 512:768]
    sc5_1, sh5_1 = _bn_affine(stat5, float(N_C5), cg1, cb1)
    h5b, stat5b = _norm_matmul(h5, sc5_1, sh5_1, cW2, 1000)
    sc5_2, sh5_2 = _bn_affine(stat5b, float(N_C5), cg2, cb2)
    c5_out = _norm_relu(h5b, sc5_2, sh5_2, 1000)

    h6 = p6[:, 512:768]
    sc6_1, sh6_1 = _bn_affine(stat6, float(N_C6), cg1, cb1)
    h6b, stat6b = _norm_matmul(h6, sc6_1, sh6_1, cW2, 1200)
    sc6_2, sh6_2 = _bn_affine(stat6b, float(N_C6), cg2, cb2)
    c6_out = _norm_relu(h6b, sc6_2, sh6_2, 1200)

    return (edge_out, c5_out, c6_out)


# speedup vs baseline: 6.5158x; 6.5158x over previous
import functools

import jax
import jax.numpy as jnp
from jax import lax
from jax.experimental import pallas as pl
from jax.experimental.compute_on import compute_on
from jax.experimental.pallas import tpu as pltpu
from jax.experimental.pallas import tpu_sc as plsc

HID = 128
N_E = 320000
N_C5 = 25000
N_C6 = 30000

CH_E = 256
NCH_E = 216
GB_E = 128
CH_V = 320
NCH_V = 1000
GB_V = 64
NW = 32


def _rup(x, m):
    return (x + m - 1) // m * m


def _sread(po_ref, i):
    return po_ref[pl.ds(i, 16)][0]


def _make_sc_scatter(nt, c, ch_rows, nch, gb, m_pad, npo):
    chp = ch_rows + 16
    trips = (nch + NW - 1) // NW
    mesh = plsc.VectorSubcoreMesh(core_axis_name="c", subcore_axis_name="s")

    @functools.partial(
        pl.kernel,
        out_type=jax.ShapeDtypeStruct((nch * ch_rows, c), jnp.float32),
        mesh=mesh,
        scratch_types=[
            pltpu.VMEM((npo,), jnp.int32),
            pltpu.VMEM((gb,), jnp.int32),
            pltpu.VMEM((gb,), jnp.int32),
            pltpu.VMEM((gb + 16,), jnp.int32),
            pltpu.VMEM((gb + 16,), jnp.int32),
            pltpu.VMEM((gb, c), jnp.float32),
            pltpu.VMEM((gb, c), jnp.float32),
            pltpu.VMEM((chp, c), jnp.float32),
            pltpu.SemaphoreType.DMA,
            pltpu.SemaphoreType.DMA,
            pltpu.SemaphoreType.DMA,
            pltpu.SemaphoreType.DMA,
            pltpu.SemaphoreType.DMA,
            pltpu.SemaphoreType.DMA,
        ],
    )
    def scatter_kernel(table, src_hbm, l2_hbm, po_hbm, out,
                       po_v, src0, src1, lb0, lb1, rows0, rows1, acc,
                       ss0, ss1, sl0, sl1, sr0, sr1):
        cid = lax.axis_index("c")
        sid = lax.axis_index("s")
        wid = sid * 2 + cid
        srcb = (src0, src1)
        lbb = (lb0, lb1)
        rowsb = (rows0, rows1)
        ssem = (ss0, ss1)
        lsem = (sl0, sl1)
        rsem = (sr0, sr1)
        zvec = jnp.zeros((16,), jnp.float32)

        pltpu.sync_copy(po_hbm, po_v)

        def process(chi):
            p0 = _sread(po_v, chi)
            p1 = _sread(po_v, chi + 1)
            bstart = pl.multiple_of(p0 // gb * gb, gb)
            ncb = (p1 - bstart + gb - 1) // gb

            def zrow(r, carry):
                for cc in range(c // 16):
                    acc[r, pl.ds(cc * 16, 16)] = zvec
                return carry

            lax.fori_loop(0, chp, zrow, 0)

            def fetch(n, q):
                ofs = pl.multiple_of(bstart + n * gb, gb)
                pltpu.async_copy(src_hbm.at[pl.ds(ofs, gb)], srcb[q],
                                 ssem[q])
                pltpu.async_copy(l2_hbm.at[pl.ds(ofs, gb)],
                                 lbb[q].at[pl.ds(0, gb)], lsem[q])

            def fetch_wait(q):
                pltpu.make_async_copy(src_hbm.at[pl.ds(0, gb)], srcb[q],
                                      ssem[q]).wait()
                pltpu.make_async_copy(l2_hbm.at[pl.ds(0, gb)],
                                      lbb[q].at[pl.ds(0, gb)],
                                      lsem[q]).wait()

            @pl.when(ncb > 0)
            def _():
                fetch(0, 0)
                fetch_wait(0)
                pltpu.async_copy(table.at[srcb[0]], rowsb[0], rsem[0])

                def batch_for(g, p):
                    q = 1 - p
                    nxt = g + 1

                    @pl.when(nxt < ncb)
                    def _():
                        fetch(nxt, q)
                    pltpu.make_async_copy(table.at[srcb[p]], rowsb[p],
                                          rsem[p]).wait()

                    @pl.when(nxt < ncb)
                    def _():
                        fetch_wait(q)
                        pltpu.async_copy(table.at[srcb[q]], rowsb[q],
                                         rsem[q])
                    base = bstart + g * gb

                    def add_row(r, carry3):
                        pos = base + r

                        @pl.when(jnp.logical_and(pos >= p0, pos < p1))
                        def _():
                            row = _sread(lbb[p], r)
                            for cc in range(c // 16):
                                sl = pl.ds(cc * 16, 16)
                                acc[row, sl] += rowsb[p][r, sl]
                        return carry3

                    lax.fori_loop(0, gb, add_row, 0)

                def batch_body(g, carry2):
                    pr = lax.rem(g, 2)

                    @pl.when(pr == 0)
                    def _():
                        batch_for(g, 0)

                    @pl.when(pr == 1)
                    def _():
                        batch_for(g, 1)
                    return carry2

                lax.fori_loop(0, ncb, batch_body, 0)

            pltpu.sync_copy(
                acc.at[pl.ds(0, ch_rows)],
                out.at[pl.ds(pl.multiple_of(chi * ch_rows, 8), ch_rows)])

        def chunk_body(i, carry):
            chi = wid + i * NW

            @pl.when(chi < nch)
            def _():
                process(chi)
            return carry

        lax.fori_loop(0, trips, chunk_body, 0)

    return scatter_kernel


def _build_entries(srcs, dsts, ch_rows, nch, padb, trash_mod):
    del trash_mod
    m = srcs.shape[0]
    with compute_on("tpu_sparsecore"):
        dsts, srcs = lax.sort_key_val(dsts.astype(jnp.int32),
                                      srcs.astype(jnp.int32))
    bounds = jnp.arange(nch + 1, dtype=jnp.int32) * ch_rows
    po_body = jnp.searchsorted(dsts, bounds, side="left").astype(jnp.int32)
    npo = _rup(nch + 1, 16) + 16
    po = jnp.concatenate(
        [po_body, jnp.full((npo - nch - 1,), m, jnp.int32)])
    m_pad = _rup(m, padb) + padb
    tail = jnp.zeros((m_pad - m,), jnp.int32)
    src_pad = jnp.concatenate([srcs, tail])
    ldst_pad = jnp.concatenate([dsts % ch_rows, tail])
    return src_pad, ldst_pad, po, m_pad, npo


def _proj_body(g, db, dir_ref, cyc_ref, w0_ref, wd_ref, wc_ref,
               out_ref, stat_ref):
    r = dir_ref.shape[0]
    x = dir_ref[...]
    rows = lax.broadcasted_iota(jnp.int32, (db, r), 1)
    cols = lax.broadcasted_iota(jnp.int32, (db, r), 0)
    msum = (rows // g == cols).astype(jnp.float32)
    d = jnp.dot(msum, x, preferred_element_type=jnp.float32)
    dsb = jnp.dot(msum.T, d, preferred_element_type=jnp.float32)
    out = (jnp.dot(x, w0_ref[...], preferred_element_type=jnp.float32)
           + jnp.dot(dsb, wd_ref[...], preferred_element_type=jnp.float32)
           + jnp.dot(cyc_ref[...], wc_ref[...],
                     preferred_element_type=jnp.float32))
    out_ref[...] = out
    h = out[:, 512:768]

    @pl.when(pl.program_id(0) == 0)
    def _():
        stat_ref[...] = jnp.zeros_like(stat_ref)
    stat_ref[0:1, :] += jnp.sum(h, axis=0, keepdims=True)
    stat_ref[1:2, :] += jnp.sum(h * h, axis=0, keepdims=True)


def _cycle_proj(dirx, cyc, w0, wd, wc, g, r, db):
    n = dirx.shape[0]
    grid = n // r
    body = functools.partial(_proj_body, g, db)
    return pl.pallas_call(
        body,
        grid=(grid,),
        in_specs=[
            pl.BlockSpec((r, HID), lambda i: (i, 0)),
            pl.BlockSpec((r, HID), lambda i: (i, 0)),
            pl.BlockSpec((HID, 768), lambda i: (0, 0)),
            pl.BlockSpec((HID, 768), lambda i: (0, 0)),
            pl.BlockSpec((HID, 768), lambda i: (0, 0)),
        ],
        out_specs=[
            pl.BlockSpec((r, 768), lambda i: (i, 0)),
            pl.BlockSpec((8, 256), lambda i: (0, 0)),
        ],
        out_shape=[
            jax.ShapeDtypeStruct((n, 768), jnp.float32),
            jax.ShapeDtypeStruct((8, 256), jnp.float32),
        ],
    )(dirx, cyc, w0, wd, wc)


def _h1_body(e_ref, v_ref, wa_ref, out_ref, stat_ref):
    h = jnp.dot(e_ref[...], wa_ref[...],
                preferred_element_type=jnp.float32) + v_ref[...]
    out_ref[...] = h

    @pl.when(pl.program_id(0) == 0)
    def _():
        stat_ref[...] = jnp.zeros_like(stat_ref)
    stat_ref[0:1, :] += jnp.sum(h, axis=0, keepdims=True)
    stat_ref[1:2, :] += jnp.sum(h * h, axis=0, keepdims=True)


def _edge_h1(edge, veff, wa, r):
    n = edge.shape[0]
    return pl.pallas_call(
        _h1_body,
        grid=(n // r,),
        in_specs=[
            pl.BlockSpec((r, HID), lambda i: (i, 0)),
            pl.BlockSpec((r, 256), lambda i: (i, 0)),
            pl.BlockSpec((HID, 256), lambda i: (0, 0)),
        ],
        out_specs=[
            pl.BlockSpec((r, 256), lambda i: (i, 0)),
            pl.BlockSpec((8, 256), lambda i: (0, 0)),
        ],
        out_shape=[
            jax.ShapeDtypeStruct((n, 256), jnp.float32),
            jax.ShapeDtypeStruct((8, 256), jnp.float32),
        ],
    )(edge, veff, wa)


def _nm_body(x_ref, sc_ref, sh_ref, w_ref, out_ref, stat_ref):
    y = jnp.maximum(x_ref[...] * sc_ref[0:1, :] + sh_ref[0:1, :], 0.0)
    h = jnp.dot(y, w_ref[...], preferred_element_type=jnp.float32)
    out_ref[...] = h

    @pl.when(pl.program_id(0) == 0)
    def _():
        stat_ref[...] = jnp.zeros_like(stat_ref)
    stat_ref[0:1, :] += jnp.sum(h, axis=0, keepdims=True)
    stat_ref[1:2, :] += jnp.sum(h * h, axis=0, keepdims=True)


def _norm_matmul(x, scale, shift, w, r):
    n, cin = x.shape
    cout = w.shape[1]
    return pl.pallas_call(
        _nm_body,
        grid=(n // r,),
        in_specs=[
            pl.BlockSpec((r, cin), lambda i: (i, 0)),
            pl.BlockSpec((8, cin), lambda i: (0, 0)),
            pl.BlockSpec((8, cin), lambda i: (0, 0)),
            pl.BlockSpec((cin, cout), lambda i: (0, 0)),
        ],
        out_specs=[
            pl.BlockSpec((r, cout), lambda i: (i, 0)),
            pl.BlockSpec((8, cout), lambda i: (0, 0)),
        ],
        out_shape=[
            jax.ShapeDtypeStruct((n, cout), jnp.float32),
            jax.ShapeDtypeStruct((8, cout), jnp.float32),
        ],
    )(x, scale, shift, w)


def _relu_body(x_ref, sc_ref, sh_ref, out_ref):
    out_ref[...] = jnp.maximum(
        x_ref[...] * sc_ref[0:1, :] + sh_ref[0:1, :], 0.0)


def _norm_relu(x, scale, shift, r):
    n, c = x.shape
    return pl.pallas_call(
        _relu_body,
        grid=(n // r,),
        in_specs=[
            pl.BlockSpec((r, c), lambda i: (i, 0)),
            pl.BlockSpec((8, c), lambda i: (0, 0)),
            pl.BlockSpec((8, c), lambda i: (0, 0)),
        ],
        out_specs=pl.BlockSpec((r, c), lambda i: (i, 0)),
        out_shape=jax.ShapeDtypeStruct((n, c), jnp.float32),
    )(x, scale, shift)


def _bn_affine(stat, n, gamma, beta):
    mu = stat[0, :] / n
    var = stat[1, :] / n - mu * mu
    scale = gamma / jnp.sqrt(var + 1e-5)
    shift = beta - mu * scale
    pad = jnp.zeros((8, scale.shape[0]), jnp.float32)
    return pad.at[0, :].set(scale), pad.at[0, :].set(shift)


def kernel(edge_rep, cycle5_rep, cycle6_rep, eW1, eg1, eb1, eW2, eg2, eb2,
           cW1, cg1, cb1, cW2, cg2, cb2, e2c5_src, e2c5_dst, e2c6_src,
           e2c6_dst, c2e5_src, c2e5_dst, c2e6_src, c2e6_dst):
    f32 = jnp.float32
    edge_rep = edge_rep.astype(f32)

    def fold(g):
        w0 = jnp.concatenate(
            [eW1[128:256], eW1[768:896], cW1[0:128]], axis=1)
        wd = jnp.concatenate(
            [eW1[256:384] + eW1[384:512] + g * eW1[512:640],
             eW1[896:1024] + eW1[1024:1152] + g * eW1[1152:1280],
             cW1[128:256] + cW1[256:384] + g * cW1[384:512]], axis=1)
        wc = jnp.concatenate(
            [eW1[640:768], eW1[1280:1408], cW1[512:640]], axis=1)
        return w0, wd, wc

    w0_5, wd_5, wc_5 = fold(5.0)
    w0_6, wd_6, wc_6 = fold(6.0)
    wa = eW1[0:128]

    src_e = jnp.concatenate([e2c5_src, e2c6_src]).astype(jnp.int32)
    dst_e = jnp.concatenate(
        [e2c5_dst, e2c6_dst + N_C5]).astype(jnp.int32)
    sp_e, l2_e, po_e, mpad_e, npo_e = _build_entries(
        src_e, dst_e, CH_E, NCH_E, GB_E, 16)
    sc1 = _make_sc_scatter(N_E, HID, CH_E, NCH_E, GB_E, mpad_e, npo_e)
    dir_all = sc1(edge_rep, sp_e, l2_e, po_e)
    dir5 = dir_all[:N_C5]
    dir6 = dir_all[N_C5:N_C5 + N_C6]

    p5, stat5 = _cycle_proj(dir5, cycle5_rep.astype(f32),
                            w0_5, wd_5, wc_5, 5, 1000, 200)
    p6, stat6 = _cycle_proj(dir6, cycle6_rep.astype(f32),
                            w0_6, wd_6, wc_6, 6, 1200, 200)
    pstack = jnp.concatenate(
        [p5[:, 0:256], p5[:, 256:512], p6[:, 0:256], p6[:, 256:512]], axis=0)

    b5c = N_C5
    b6b = 2 * N_C5
    b6c = 2 * N_C5 + N_C6
    s5 = c2e5_src.astype(jnp.int32)
    d5 = c2e5_dst.astype(jnp.int32)
    s6 = c2e6_src.astype(jnp.int32)
    d6 = c2e6_dst.astype(jnp.int32)
    src_v = jnp.concatenate(
        [s5, s5 + b5c, s5 + b5c, s6 + b6b, s6 + b6c, s6 + b6c])
    dst_v = jnp.concatenate(
        [d5, d5 & ~1, d5 | 1, d6, d6 & ~1, d6 | 1])
    sp_v, l2_v, po_v, mpad_v, npo_v = _build_entries(
        src_v, dst_v, CH_V, NCH_V, GB_V, 16)
    sc2 = _make_sc_scatter(2 * (N_C5 + N_C6), 256, CH_V, NCH_V, GB_V,
                           mpad_v, npo_v)
    veff = sc2(pstack, sp_v, l2_v, po_v)

    h1, stat_h1 = _edge_h1(edge_rep, veff, wa, 1000)
    sc_1, sh_1 = _bn_affine(stat_h1, float(N_E), eg1, eb1)
    h2, stat_h2 = _norm_matmul(h1, sc_1, sh_1, eW2, 1000)
    sc_2, sh_2 = _bn_affine(stat_h2, float(N_E), eg2, eb2)
    edge_out = _norm_relu(h2, sc_2, sh_2, 1000)

    h5 = p5[:, 512:768]
    sc5_1, sh5_1 = _bn_affine(stat5, float(N_C5), cg1, cb1)
    h5b, stat5b = _norm_matmul(h5, sc5_1, sh5_1, cW2, 1000)
    sc5_2, sh5_2 = _bn_affine(stat5b, float(N_C5), cg2, cb2)
    c5_out = _norm_relu(h5b, sc5_2, sh5_2, 1000)

    h6 = p6[:, 512:768]
    sc6_1, sh6_1 = _bn_affine(stat6, float(N_C6), cg1, cb1)
    h6b, stat6b = _norm_matmul(h6, sc6_1, sh6_1, cW2, 1200)
    sc6_2, sh6_2 = _bn_affine(stat6b, float(N_C6), cg2, cb2)
    c6_out = _norm_relu(h6b, sc6_2, sh6_2, 1200)

    return (edge_out, c5_out, c6_out)
